# baseline (speedup 1.0000x reference)
import functools

import jax
import jax.numpy as jnp
import numpy as np
from jax import lax
from jax.experimental import pallas as pl
from jax.experimental.pallas import tpu as pltpu
from jax.experimental.pallas import tpu_sc as plsc

N = 10000
D = 128
G = 512
GLOBAL_DIM = 128
H = 256

NUM_RG = 4
NUM_FG = 8
ROWS_PER = 2512
FPW = D // NUM_FG
BLK = 16

_mesh = plsc.VectorSubcoreMesh(core_axis_name="c", subcore_axis_name="s")


@functools.partial(
    pl.kernel,
    mesh=_mesh,
    compiler_params=pltpu.CompilerParams(
        use_tc_tiling_on_sc=False, needs_layout_passes=False),
    out_type=[
        jax.ShapeDtypeStruct((NUM_RG, G, D), jnp.float32),
        jax.ShapeDtypeStruct((NUM_RG, G, D), jnp.float32),
        jax.ShapeDtypeStruct((NUM_RG, G, D), jnp.float32),
        jax.ShapeDtypeStruct((NUM_RG, G, D), jnp.float32),
        jax.ShapeDtypeStruct((NUM_RG, 520, 128), jnp.float32),
    ],
    scratch_types=[
        pltpu.VMEM((ROWS_PER, FPW), jnp.float32),
        pltpu.VMEM((ROWS_PER + 16,), jnp.int32),
        pltpu.VMEM((G, FPW), jnp.float32),
        pltpu.VMEM((G, FPW), jnp.float32),
        pltpu.VMEM((G, FPW), jnp.float32),
        pltpu.VMEM((G, FPW), jnp.float32),
        pltpu.VMEM((520, 16), jnp.float32),
        pltpu.SemaphoreType.DMA,
        pltpu.SemaphoreType.DMA,
        pltpu.SemaphoreType.DMA,
        pltpu.SemaphoreType.DMA,
    ],
)
def _sc_aggregate(x_hbm, batch_hbm,
                  sums_hbm, sq_hbm, mx_hbm, mn_hbm, cnt_hbm,
                  xv, bv, accs, accq, accmx, accmn, accc,
                  sem_b, sem_x, sem_x2, sem_o):
    cix = lax.axis_index("c")
    six = lax.axis_index("s")
    wid = six * 2 + cix
    fg = wid // NUM_RG
    rg = wid % NUM_RG
    r0 = rg * 2496
    f0 = fg * FPW
    nblk = jnp.where(rg == NUM_RG - 1, 2512 // BLK, 2496 // BLK)

    hb = pltpu.async_copy(
        batch_hbm.at[pl.ds(r0, ROWS_PER)], bv.at[pl.ds(0, ROWS_PER)], sem_b)
    hx = pltpu.async_copy(
        x_hbm.at[pl.ds(r0, 1264), pl.ds(f0, FPW)],
        xv.at[pl.ds(0, 1264)], sem_x)
    hx2 = pltpu.async_copy(
        x_hbm.at[pl.ds(r0 + 1264, ROWS_PER - 1264), pl.ds(f0, FPW)],
        xv.at[pl.ds(1264, ROWS_PER - 1264)], sem_x2)

    lanes = lax.iota(jnp.int32, 16)
    zero = jnp.zeros((16,), jnp.float32)
    neg = jnp.full((16,), -jnp.inf, jnp.float32)
    pos = jnp.full((16,), jnp.inf, jnp.float32)

    hb.wait()
    first = bv[pl.ds(0, 16)][0]
    nrows = jnp.where(rg == NUM_RG - 1, 2512, 2496)
    last0 = bv[pl.ds(nrows - 16, 16)][15]

    def zbody(i, _):
        plsc.store_scatter(accc, [jnp.full((16,), i, jnp.int32), lanes], zero)
        return 0
    lax.fori_loop(first, last0 + 1, zbody, 0)
    hx.wait()

    def flush(cur, cnt, vs, vq, vmx, vmn):
        i0 = jnp.full((16,), cur, jnp.int32)
        plsc.store_scatter(accs, [i0, lanes], vs)
        plsc.store_scatter(accq, [i0, lanes], vq)
        plsc.store_scatter(accmx, [i0, lanes], vmx)
        plsc.store_scatter(accmn, [i0, lanes], vmn)
        plsc.store_scatter(accc, [i0, lanes], jnp.full((16,), cnt, jnp.float32))

    def step(seg, v, carry):
        cur, cnt, vs, vq, vmx, vmn = carry
        ch = seg != cur
        chv = jnp.full((16,), ch)
        i0 = jnp.full((16,), cur, jnp.int32)
        plsc.store_scatter(accs, [i0, lanes], vs, mask=chv)
        plsc.store_scatter(accq, [i0, lanes], vq, mask=chv)
        plsc.store_scatter(accmx, [i0, lanes], vmx, mask=chv)
        plsc.store_scatter(accmn, [i0, lanes], vmn, mask=chv)
        plsc.store_scatter(accc, [i0, lanes],
                           jnp.full((16,), cnt, jnp.float32), mask=chv)
        vv = v * v
        return (seg,
                jnp.where(ch, 1.0, cnt + 1.0),
                jnp.where(ch, v, vs + v),
                jnp.where(ch, vv, vq + vv),
                jnp.where(ch, v, jnp.maximum(vmx, v)),
                jnp.where(ch, v, jnp.minimum(vmn, v)))

    def blk_body(b, carry):
        lb = b * BLK
        segv = bv[pl.ds(lb, 16)]
        for j in range(BLK):
            carry = step(segv[j], xv[lb + j], carry)
        return carry

    carry = lax.fori_loop(
        0, 1264 // BLK, blk_body, (first, 0.0, zero, zero, neg, pos))
    hx2.wait()
    carry = lax.fori_loop(1264 // BLK, nblk, blk_body, carry)
    flush(*carry)
    last = carry[0]

    h1 = pltpu.async_copy(accs, sums_hbm.at[rg, :, pl.ds(f0, FPW)], sem_o)
    h2 = pltpu.async_copy(accq, sq_hbm.at[rg, :, pl.ds(f0, FPW)], sem_o)
    h3 = pltpu.async_copy(accmx, mx_hbm.at[rg, :, pl.ds(f0, FPW)], sem_o)
    h4 = pltpu.async_copy(accmn, mn_hbm.at[rg, :, pl.ds(f0, FPW)], sem_o)
    h1.wait()
    h2.wait()
    h3.wait()
    h4.wait()

    @pl.when(fg == 0)
    def _():
        accc[512] = jnp.full((16,), first.astype(jnp.float32))
        accc[513] = jnp.full((16,), last.astype(jnp.float32))
        pltpu.sync_copy(accc, cnt_hbm.at[rg, :, pl.ds(0, 16)])


def _tc_finish_body(sums_ref, sq_ref, mx_ref, mn_ref, cnt_ref,
                    u_ref, W1_ref, b1_ref, g_ref, be_ref, W2_ref, b2_ref,
                    out_ref):
    segs = lax.broadcasted_iota(jnp.int32, (G, 1), 0).astype(jnp.float32)

    s_sum = jnp.zeros((G, D), jnp.float32)
    s_q = jnp.zeros((G, D), jnp.float32)
    s_mx = jnp.full((G, D), -jnp.inf, jnp.float32)
    s_mn = jnp.full((G, D), jnp.inf, jnp.float32)
    counts = jnp.zeros((G, 1), jnp.float32)
    for i in range(NUM_RG):
        rf = cnt_ref[i, 512:513, 0:1]
        rl = cnt_ref[i, 513:514, 0:1]
        valid = (segs >= rf) & (segs <= rl)
        s_sum = s_sum + jnp.where(valid, sums_ref[i], 0.0)
        s_q = s_q + jnp.where(valid, sq_ref[i], 0.0)
        s_mx = jnp.maximum(s_mx, jnp.where(valid, mx_ref[i], -jnp.inf))
        s_mn = jnp.minimum(s_mn, jnp.where(valid, mn_ref[i], jnp.inf))
        counts = counts + jnp.where(valid, cnt_ref[i, 0:512, 0:1], 0.0)

    c1 = jnp.maximum(counts, 1.0)
    mean = s_sum / c1
    mean2 = s_q / c1
    var = jnp.maximum(mean2 - mean * mean, 0.0)
    std = jnp.sqrt(var + 1e-5)
    present = counts > 0.0
    mean = jnp.where(present, mean, 0.0)
    std = jnp.where(present, std, float(np.sqrt(1e-5)))
    s_mx = jnp.where(present, s_mx, 0.0)
    s_mn = jnp.where(present, s_mn, 0.0)

    big = jnp.concatenate([u_ref[...], mean, std, s_mx, s_mn], axis=1)
    h = jnp.dot(big, W1_ref[...], preferred_element_type=jnp.float32)
    h = h + b1_ref[...]
    alpha = 1.6732632423543772
    scale = 1.0507009873554805
    h = scale * jnp.where(h > 0, h, alpha * (jnp.exp(h) - 1.0))
    mu = jnp.mean(h, axis=1, keepdims=True)
    varh = jnp.mean((h - mu) ** 2, axis=1, keepdims=True)
    h = (h - mu) / jnp.sqrt(varh + 1e-5) * g_ref[...] + be_ref[...]
    out_ref[...] = jnp.dot(h, W2_ref[...],
                           preferred_element_type=jnp.float32) + b2_ref[...]


_tc_finish = pl.pallas_call(
    _tc_finish_body,
    out_shape=jax.ShapeDtypeStruct((G, GLOBAL_DIM), jnp.float32),
)


def kernel(x, edge_index, edge_attr, u, batch, W1, b1, gamma, beta, W2, b2):
    del edge_index, edge_attr
    sums, sq, mx, mn, cnt = _sc_aggregate(
        x.astype(jnp.float32), batch.astype(jnp.int32))
    return _tc_finish(sums, sq, mx, mn, cnt,
                      u, W1, b1.reshape(1, H), gamma.reshape(1, H),
                      beta.reshape(1, H), W2, b2.reshape(1, GLOBAL_DIM))


# speedup vs baseline: 1.0142x; 1.0142x over previous
import functools

import jax
import jax.numpy as jnp
import numpy as np
from jax import lax
from jax.experimental import pallas as pl
from jax.experimental.pallas import tpu as pltpu
from jax.experimental.pallas import tpu_sc as plsc

N = 10000
D = 128
G = 512
GLOBAL_DIM = 128
H = 256

NUM_RG = 4
NUM_FG = 8
ROWS_PER = 2512
FPW = D // NUM_FG
BLK = 16

_mesh = plsc.VectorSubcoreMesh(core_axis_name="c", subcore_axis_name="s")


@functools.partial(
    pl.kernel,
    mesh=_mesh,
    compiler_params=pltpu.CompilerParams(
        use_tc_tiling_on_sc=False, needs_layout_passes=False),
    out_type=[
        jax.ShapeDtypeStruct((NUM_RG, G, D), jnp.float32),
        jax.ShapeDtypeStruct((NUM_RG, G, D), jnp.float32),
        jax.ShapeDtypeStruct((NUM_RG, G, D), jnp.float32),
        jax.ShapeDtypeStruct((NUM_RG, G, D), jnp.float32),
        jax.ShapeDtypeStruct((NUM_RG, 520, 128), jnp.float32),
    ],
    scratch_types=[
        pltpu.VMEM((ROWS_PER, FPW), jnp.float32),
        pltpu.VMEM((ROWS_PER + 16,), jnp.int32),
        pltpu.VMEM((G, FPW), jnp.float32),
        pltpu.VMEM((G, FPW), jnp.float32),
        pltpu.VMEM((G, FPW), jnp.float32),
        pltpu.VMEM((G, FPW), jnp.float32),
        pltpu.VMEM((520, 16), jnp.float32),
        pltpu.SemaphoreType.DMA,
        pltpu.SemaphoreType.DMA,
        pltpu.SemaphoreType.DMA,
    ],
)
def _sc_aggregate(x_hbm, batch_hbm,
                  sums_hbm, sq_hbm, mx_hbm, mn_hbm, cnt_hbm,
                  xv, bv, accs, accq, accmx, accmn, accc,
                  sem_b, sem_x, sem_o):
    cix = lax.axis_index("c")
    six = lax.axis_index("s")
    wid = six * 2 + cix
    fg = wid // NUM_RG
    rg = wid % NUM_RG
    r0 = rg * 2496
    f0 = fg * FPW
    nblk = jnp.where(rg == NUM_RG - 1, 2512 // BLK, 2496 // BLK)

    hb = pltpu.async_copy(
        batch_hbm.at[pl.ds(r0, ROWS_PER)], bv.at[pl.ds(0, ROWS_PER)], sem_b)
    hx = pltpu.async_copy(
        x_hbm.at[pl.ds(r0, ROWS_PER), pl.ds(f0, FPW)], xv, sem_x)

    lanes = lax.iota(jnp.int32, 16)
    zero = jnp.zeros((16,), jnp.float32)
    neg = jnp.full((16,), -jnp.inf, jnp.float32)
    pos = jnp.full((16,), jnp.inf, jnp.float32)

    hb.wait()
    first = bv[pl.ds(0, 16)][0]
    nrows = jnp.where(rg == NUM_RG - 1, 2512, 2496)
    last0 = bv[pl.ds(nrows - 16, 16)][15]

    def zbody(i, _):
        plsc.store_scatter(accc, [jnp.full((16,), i, jnp.int32), lanes], zero)
        return 0
    lax.fori_loop(first, last0 + 1, zbody, 0)
    hx.wait()

    def flush(cur, cnt, vs, vq, vmx, vmn):
        i0 = jnp.full((16,), cur, jnp.int32)
        plsc.store_scatter(accs, [i0, lanes], vs)
        plsc.store_scatter(accq, [i0, lanes], vq)
        plsc.store_scatter(accmx, [i0, lanes], vmx)
        plsc.store_scatter(accmn, [i0, lanes], vmn)
        plsc.store_scatter(accc, [i0, lanes], jnp.full((16,), cnt, jnp.float32))

    def step(seg, v, carry):
        cur, cnt, vs, vq, vmx, vmn = carry
        ch = seg != cur
        chv = jnp.full((16,), ch)
        i0 = jnp.full((16,), cur, jnp.int32)
        plsc.store_scatter(accs, [i0, lanes], vs, mask=chv)
        plsc.store_scatter(accq, [i0, lanes], vq, mask=chv)
        plsc.store_scatter(accmx, [i0, lanes], vmx, mask=chv)
        plsc.store_scatter(accmn, [i0, lanes], vmn, mask=chv)
        plsc.store_scatter(accc, [i0, lanes],
                           jnp.full((16,), cnt, jnp.float32), mask=chv)
        vv = v * v
        return (seg,
                jnp.where(ch, 1.0, cnt + 1.0),
                jnp.where(ch, v, vs + v),
                jnp.where(ch, vv, vq + vv),
                jnp.where(ch, v, jnp.maximum(vmx, v)),
                jnp.where(ch, v, jnp.minimum(vmn, v)))

    def blk_body(b, carry):
        lb = b * BLK
        segv = bv[pl.ds(lb, 16)]
        for j in range(BLK):
            carry = step(segv[j], xv[lb + j], carry)
        return carry

    carry = lax.fori_loop(
        0, nblk, blk_body, (first, 0.0, zero, zero, neg, pos))
    flush(*carry)
    last = carry[0]

    h1 = pltpu.async_copy(accs, sums_hbm.at[rg, :, pl.ds(f0, FPW)], sem_o)
    h2 = pltpu.async_copy(accq, sq_hbm.at[rg, :, pl.ds(f0, FPW)], sem_o)
    h3 = pltpu.async_copy(accmx, mx_hbm.at[rg, :, pl.ds(f0, FPW)], sem_o)
    h4 = pltpu.async_copy(accmn, mn_hbm.at[rg, :, pl.ds(f0, FPW)], sem_o)
    h1.wait()
    h2.wait()
    h3.wait()
    h4.wait()

    @pl.when(fg == 0)
    def _():
        accc[512] = jnp.full((16,), first.astype(jnp.float32))
        accc[513] = jnp.full((16,), last.astype(jnp.float32))
        pltpu.sync_copy(accc, cnt_hbm.at[rg, :, pl.ds(0, 16)])


def _tc_finish_body(sums_ref, sq_ref, mx_ref, mn_ref, cnt_ref,
                    u_ref, W1_ref, b1_ref, g_ref, be_ref, W2_ref, b2_ref,
                    out_ref):
    segs = lax.broadcasted_iota(jnp.int32, (G, 1), 0).astype(jnp.float32)

    s_sum = jnp.zeros((G, D), jnp.float32)
    s_q = jnp.zeros((G, D), jnp.float32)
    s_mx = jnp.full((G, D), -jnp.inf, jnp.float32)
    s_mn = jnp.full((G, D), jnp.inf, jnp.float32)
    counts = jnp.zeros((G, 1), jnp.float32)
    for i in range(NUM_RG):
        rf = cnt_ref[i, 512:513, 0:1]
        rl = cnt_ref[i, 513:514, 0:1]
        valid = (segs >= rf) & (segs <= rl)
        s_sum = s_sum + jnp.where(valid, sums_ref[i], 0.0)
        s_q = s_q + jnp.where(valid, sq_ref[i], 0.0)
        s_mx = jnp.maximum(s_mx, jnp.where(valid, mx_ref[i], -jnp.inf))
        s_mn = jnp.minimum(s_mn, jnp.where(valid, mn_ref[i], jnp.inf))
        counts = counts + jnp.where(valid, cnt_ref[i, 0:512, 0:1], 0.0)

    c1 = jnp.maximum(counts, 1.0)
    mean = s_sum / c1
    mean2 = s_q / c1
    var = jnp.maximum(mean2 - mean * mean, 0.0)
    std = jnp.sqrt(var + 1e-5)
    present = counts > 0.0
    mean = jnp.where(present, mean, 0.0)
    std = jnp.where(present, std, float(np.sqrt(1e-5)))
    s_mx = jnp.where(present, s_mx, 0.0)
    s_mn = jnp.where(present, s_mn, 0.0)

    big = jnp.concatenate([u_ref[...], mean, std, s_mx, s_mn], axis=1)
    h = jnp.dot(big, W1_ref[...], preferred_element_type=jnp.float32)
    h = h + b1_ref[...]
    alpha = 1.6732632423543772
    scale = 1.0507009873554805
    h = scale * jnp.where(h > 0, h, alpha * (jnp.exp(h) - 1.0))
    mu = jnp.mean(h, axis=1, keepdims=True)
    varh = jnp.mean((h - mu) ** 2, axis=1, keepdims=True)
    h = (h - mu) / jnp.sqrt(varh + 1e-5) * g_ref[...] + be_ref[...]
    out_ref[...] = jnp.dot(h, W2_ref[...],
                           preferred_element_type=jnp.float32) + b2_ref[...]


_tc_finish = pl.pallas_call(
    _tc_finish_body,
    out_shape=jax.ShapeDtypeStruct((G, GLOBAL_DIM), jnp.float32),
)


def kernel(x, edge_index, edge_attr, u, batch, W1, b1, gamma, beta, W2, b2):
    del edge_index, edge_attr
    sums, sq, mx, mn, cnt = _sc_aggregate(
        x.astype(jnp.float32), batch.astype(jnp.int32))
    return _tc_finish(sums, sq, mx, mn, cnt,
                      u, W1, b1.reshape(1, H), gamma.reshape(1, H),
                      beta.reshape(1, H), W2, b2.reshape(1, GLOBAL_DIM))
